# baseline (speedup 1.0000x reference)
import functools

import jax
import jax.numpy as jnp
from jax import lax
from jax.experimental import pallas as pl
from jax.experimental.pallas import tpu as pltpu
from jax.experimental.pallas import tpu_sc as plsc

D = 32
NC = 2
NS = 16
NW = NC * NS
NB = 5


def _sine_pos(seq_len, d, max_wavelength=10000.0):
    position = jnp.arange(seq_len, dtype=jnp.float32)
    min_freq = 1.0 / max_wavelength
    timescales = jnp.power(
        min_freq,
        (2.0 * (jnp.arange(d, dtype=jnp.float32) // 2)) / float(d),
    )
    angles = position[:, None] * timescales[None, :]
    cos_mask = (jnp.arange(d) % 2).astype(jnp.float32)
    sin_mask = 1.0 - cos_mask
    return jnp.sin(angles) * sin_mask + jnp.cos(angles) * cos_mask


def kernel(input, table):
    B, S = input.shape
    V, d = table.shape
    N = B * S
    BW = B // NW
    assert BW == 128 and d == D and (S - NB) % NB == 0

    pos = _sine_pos(S, d)
    idx_t = input.T

    mesh = plsc.VectorSubcoreMesh(core_axis_name="c", subcore_axis_name="s")

    RB = 256
    G1 = V // RB
    TAIL = V - G1 * RB
    n_hi = G1 - (G1 // NW) * NW

    @functools.partial(
        pl.kernel,
        mesh=mesh,
        compiler_params=pltpu.CompilerParams(use_tc_tiling_on_sc=True,
                                             needs_layout_passes=False),
        out_type=jax.ShapeDtypeStruct((V // 4, 128), jnp.float32),
        scratch_types=[
            [pltpu.VMEM((RB, D), jnp.float32) for _ in range(2)],
            [pltpu.VMEM((RB // 4, 128), jnp.float32) for _ in range(2)],
            [pltpu.SemaphoreType.DMA for _ in range(2)],
            [pltpu.SemaphoreType.DMA for _ in range(2)],
        ],
    )
    def sc_fmt(tab_hbm, out_hbm, bi, bo, isem, osem):
        wid = lax.axis_index("s") * NC + lax.axis_index("c")
        n1 = jnp.where(wid < n_hi, G1 // NW + 1, G1 // NW)

        def fire_in(b, t):
            g = wid + NW * t
            pltpu.async_copy(tab_hbm.at[pl.ds(g * RB, RB)], bi[b], isem[b])

        def wait_in(b):
            pltpu.make_async_copy(tab_hbm.at[pl.ds(0, RB)], bi[b],
                                  isem[b]).wait()

        def fire_out(b, t):
            g = wid + NW * t
            pltpu.async_copy(bo[b], out_hbm.at[pl.ds(g * (RB // 4), RB // 4)],
                             osem[b])

        def wait_out1(b):
            pltpu.make_async_copy(bo[b], out_hbm.at[pl.ds(0, RB // 4)],
                                  osem[b]).wait()

        def densify(b, nrow):
            @plsc.parallel_loop(0, nrow, step=1, unroll=8)
            def _c(v):
                q = v // 4
                rm = lax.rem(v, 4) * 32
                bo[b][q, pl.ds(rm, 16)] = bi[b][v, pl.ds(0, 16)]
                bo[b][q, pl.ds(rm + 16, 16)] = bi[b][v, pl.ds(16, 16)]

        for b in range(2):
            @pl.when(b < n1)
            def _():
                fire_in(b, jnp.int32(b))

        def floop(p, _):
            for b in range(2):
                t = 2 * p + b

                @pl.when(t < n1)
                def _():
                    wait_in(b)

                    @pl.when(t >= 2)
                    def _():
                        wait_out1(b)

                    densify(b, RB)
                    fire_out(b, t)

                    @pl.when(t + 2 < n1)
                    def _():
                        fire_in(b, t + 2)
            return 0

        lax.fori_loop(0, (G1 // NW + 2) // 2, floop, 0)
        for b in range(2):
            wait_out1(b)

        @pl.when(wid == NW - 1)
        def _():
            pltpu.sync_copy(tab_hbm.at[pl.ds(G1 * RB, TAIL)],
                            bi[0].at[pl.ds(0, TAIL)])
            densify(0, TAIL)
            pltpu.sync_copy(bo[0].at[pl.ds(0, TAIL // 4)],
                            out_hbm.at[pl.ds(G1 * (RB // 4), TAIL // 4)])

    tab4 = sc_fmt(table)
    tab_lin = tab4.reshape(V, d)

    @functools.partial(
        pl.kernel,
        mesh=mesh,
        compiler_params=pltpu.CompilerParams(use_tc_tiling_on_sc=False, needs_layout_passes=False),
        out_type=jax.ShapeDtypeStruct((S, D // 8, NW, 8, BW), jnp.float32),
        scratch_types=[
            pltpu.VMEM((S, BW), jnp.int32),
            pltpu.VMEM((S, D), jnp.float32),
            [pltpu.VMEM((BW, D), jnp.float32) for _ in range(NB)],
            [pltpu.VMEM((D, BW + 1), jnp.float32) for _ in range(NB)],
            [pltpu.SemaphoreType.DMA for _ in range(NB)],
            [pltpu.SemaphoreType.DMA for _ in range(NB)],
        ],
    )
    def sc_embed(idx_hbm, tab_hbm, pos_hbm, out_hbm, idx_v, pos_v, rows,
                 stage, gsem, osem):
        wid = lax.axis_index("s") * NC + lax.axis_index("c")
        pltpu.sync_copy(idx_hbm.at[pl.ds(0, S), pl.ds(BW * wid, BW)], idx_v)
        pltpu.sync_copy(pos_hbm, pos_v)

        iota = lax.iota(jnp.int32, 16)
        iota16 = iota + 16

        def fire_gather(b, s):
            pltpu.async_copy(tab_hbm.at[idx_v.at[s]], rows[b], gsem[b])

        def wait_gather(b):
            pltpu.make_async_copy(tab_hbm.at[pl.ds(0, BW)], rows[b],
                                  gsem[b]).wait()

        def fire_out(b, s):
            for tr in range(D // 8):
                pltpu.async_copy(
                    stage[b].at[pl.ds(8 * tr, 8), pl.ds(0, BW)],
                    out_hbm.at[s, tr, wid], osem[b])

        def wait_out(b):
            for tr in range(D // 8):
                pltpu.make_async_copy(
                    stage[b].at[pl.ds(8 * tr, 8), pl.ds(0, BW)],
                    out_hbm.at[0, 0, 0], osem[b]).wait()

        def transpose_add(b, s):
            p0 = pos_v[s, pl.ds(0, 16)]
            p1 = pos_v[s, pl.ds(16, 16)]

            @plsc.parallel_loop(0, BW, step=1, unroll=8)
            def _t(l):
                base = jnp.full((16,), l, jnp.int32)
                v0 = rows[b][l, pl.ds(0, 16)] + p0
                v1 = rows[b][l, pl.ds(16, 16)] + p1
                plsc.store_scatter(stage[b], [iota, base], v0)
                plsc.store_scatter(stage[b], [iota16, base], v1)

        for b in range(NB):
            fire_gather(b, jnp.int32(b))

        def pipe(p, _):
            for b in range(NB):
                s = p * NB + b
                wait_gather(b)

                @pl.when(p > 0)
                def _():
                    wait_out(b)

                transpose_add(b, s)
                fire_out(b, s)
                fire_gather(b, s + NB)
            return 0

        lax.fori_loop(0, (S - NB) // NB, pipe, 0)

        for b in range(NB):
            s = jnp.int32(S - NB + b)
            wait_gather(b)
            wait_out(b)
            transpose_add(b, s)
            fire_out(b, s)
        for b in range(NB):
            wait_out(b)

    out_lin = sc_embed(idx_t, tab_lin, pos)
    return out_lin.transpose(2, 4, 0, 1, 3).reshape(B, S, D)


# speedup vs baseline: 1.0422x; 1.0422x over previous
import functools

import jax
import jax.numpy as jnp
from jax import lax
from jax.experimental import pallas as pl
from jax.experimental.pallas import tpu as pltpu
from jax.experimental.pallas import tpu_sc as plsc

D = 32
NC = 2
NS = 16
NW = NC * NS
NB = 8


def _sine_pos(seq_len, d, max_wavelength=10000.0):
    position = jnp.arange(seq_len, dtype=jnp.float32)
    min_freq = 1.0 / max_wavelength
    timescales = jnp.power(
        min_freq,
        (2.0 * (jnp.arange(d, dtype=jnp.float32) // 2)) / float(d),
    )
    angles = position[:, None] * timescales[None, :]
    cos_mask = (jnp.arange(d) % 2).astype(jnp.float32)
    sin_mask = 1.0 - cos_mask
    return jnp.sin(angles) * sin_mask + jnp.cos(angles) * cos_mask


def kernel(input, table):
    B, S = input.shape
    V, d = table.shape
    N = B * S
    BW = B // NW
    assert BW == 128 and d == D and (S - NB) % NB == 0

    pos = _sine_pos(S, d)
    idx_t = input.T

    mesh = plsc.VectorSubcoreMesh(core_axis_name="c", subcore_axis_name="s")

    @functools.partial(
        pl.kernel,
        mesh=mesh,
        compiler_params=pltpu.CompilerParams(use_tc_tiling_on_sc=False, needs_layout_passes=False),
        out_type=jax.ShapeDtypeStruct((S, D // 8, NW, 8, BW), jnp.float32),
        scratch_types=[
            pltpu.VMEM((S, BW), jnp.int32),
            pltpu.VMEM((S, D), jnp.float32),
            [pltpu.VMEM((BW, D), jnp.float32) for _ in range(NB)],
            [pltpu.VMEM((D, BW + 1), jnp.float32) for _ in range(NB)],
            [pltpu.SemaphoreType.DMA for _ in range(NB)],
            [pltpu.SemaphoreType.DMA for _ in range(NB)],
        ],
    )
    def sc_embed(idx_hbm, tab_hbm, pos_hbm, out_hbm, idx_v, pos_v, rows,
                 stage, gsem, osem):
        wid = lax.axis_index("s") * NC + lax.axis_index("c")
        pltpu.sync_copy(idx_hbm.at[pl.ds(0, S), pl.ds(BW * wid, BW)], idx_v)
        pltpu.sync_copy(pos_hbm, pos_v)

        iota = lax.iota(jnp.int32, 16)
        iota16 = iota + 16

        def fire_gather(b, s):
            pltpu.async_copy(tab_hbm.at[idx_v.at[s]], rows[b], gsem[b])

        def wait_gather(b):
            pltpu.make_async_copy(tab_hbm.at[pl.ds(0, BW)], rows[b],
                                  gsem[b]).wait()

        def fire_out(b, s):
            for tr in range(D // 8):
                pltpu.async_copy(
                    stage[b].at[pl.ds(8 * tr, 8), pl.ds(0, BW)],
                    out_hbm.at[s, tr, wid], osem[b])

        def wait_out(b):
            for tr in range(D // 8):
                pltpu.make_async_copy(
                    stage[b].at[pl.ds(8 * tr, 8), pl.ds(0, BW)],
                    out_hbm.at[0, 0, 0], osem[b]).wait()

        def transpose_add(b, s):
            p0 = pos_v[s, pl.ds(0, 16)]
            p1 = pos_v[s, pl.ds(16, 16)]

            @plsc.parallel_loop(0, BW, step=1, unroll=8)
            def _t(l):
                base = jnp.full((16,), l, jnp.int32)
                v0 = rows[b][l, pl.ds(0, 16)] + p0
                v1 = rows[b][l, pl.ds(16, 16)] + p1
                plsc.store_scatter(stage[b], [iota, base], v0)
                plsc.store_scatter(stage[b], [iota16, base], v1)

        for b in range(NB):
            fire_gather(b, jnp.int32(b))

        def pipe(p, _):
            for b in range(NB):
                s = p * NB + b
                wait_gather(b)

                @pl.when(p > 0)
                def _():
                    wait_out(b)

                transpose_add(b, s)
                fire_out(b, s)
                fire_gather(b, s + NB)
            return 0

        lax.fori_loop(0, (S - NB) // NB, pipe, 0)

        for b in range(NB):
            s = jnp.int32(S - NB + b)
            wait_gather(b)
            wait_out(b)
            transpose_add(b, s)
            fire_out(b, s)
        for b in range(NB):
            wait_out(b)

    out_lin = sc_embed(idx_t, table, pos)
    return out_lin.transpose(2, 4, 0, 1, 3).reshape(B, S, D)
